# baseline (speedup 1.0000x reference)
import functools

import jax
import jax.numpy as jnp
from jax import lax
from jax.experimental import pallas as pl
from jax.experimental.pallas import tpu as pltpu
from jax.experimental.pallas import tpu_sc as plsc

B = 16384
D = 500
V = 100000
H1 = 64
NC = 2
NS = 16
NW = NC * NS
KMAX = (D + NW - 1) // NW
OC = 4096
NOC = B // OC


@functools.cache
def _make_sc_gather():
    mesh = plsc.VectorSubcoreMesh(core_axis_name="c", subcore_axis_name="s")

    @functools.partial(
        pl.kernel,
        mesh=mesh,
        out_type=(
            jax.ShapeDtypeStruct((D, B), jnp.float32),
            jax.ShapeDtypeStruct((D, B), jnp.float32),
        ),
        scratch_types=[
            pltpu.VMEM((B,), jnp.int32),
            pltpu.VMEM((V,), jnp.float32),
            pltpu.VMEM((2, OC), jnp.float32),
            pltpu.SemaphoreType.DMA,
            pltpu.SemaphoreType.DMA,
            pltpu.SemaphoreType.DMA,
        ],
        compiler_params=pltpu.CompilerParams(needs_layout_passes=False),
    )
    def _sc_gather(uid_hbm, iid_hbm, utabT_hbm, itabT_hbm, uoutT_hbm,
                   ioutT_hbm, idx_v, row_v, out_v, rsem, osem, xsem):
        wid = lax.axis_index("s") * NC + lax.axis_index("c")

        def one_table(idx_hbm, tabT_hbm, outT_hbm):
            pltpu.sync_copy(idx_hbm, idx_v)
            for k in range(KMAX):
                d = wid + NW * k

                @pl.when(d < D)
                def _():
                    pltpu.async_copy(tabT_hbm.at[d], row_v, rsem).wait()
                    for c in range(NOC):
                        buf = c % 2
                        if c >= 2:
                            pltpu.make_async_copy(
                                out_v.at[buf], outT_hbm.at[0, pl.ds(0, OC)],
                                osem).wait()

                        def gather16(v, _):
                            iv = idx_v[pl.ds(c * OC + v * 16, 16)]
                            out_v[buf, pl.ds(v * 16, 16)] = plsc.load_gather(
                                row_v, [iv])
                            return 0

                        lax.fori_loop(0, OC // 16, gather16, 0, unroll=8)
                        pltpu.async_copy(
                            out_v.at[buf], outT_hbm.at[d, pl.ds(c * OC, OC)],
                            osem)
                    pltpu.make_async_copy(
                        out_v.at[0], outT_hbm.at[0, pl.ds(0, OC)], osem).wait()
                    pltpu.make_async_copy(
                        out_v.at[1], outT_hbm.at[0, pl.ds(0, OC)], osem).wait()

        one_table(uid_hbm, utabT_hbm, uoutT_hbm)
        one_table(iid_hbm, itabT_hbm, ioutT_hbm)

    return _sc_gather


def _tc_dense_body(u_ref, i_ref, wu_ref, wi_ref, bu_ref, bi_ref, wp_ref,
                   bp_ref, o_ref):
    dn = (((0,), (0,)), ((), ()))
    u = lax.dot_general(u_ref[...], wu_ref[...], dn,
                        preferred_element_type=jnp.float32)
    u = jnp.maximum(u + bu_ref[...], 0.0)
    i = lax.dot_general(i_ref[...], wi_ref[...], dn,
                        preferred_element_type=jnp.float32)
    i = jnp.maximum(i + bi_ref[...], 0.0)
    r = jnp.dot(u, wp_ref[:H1, :], preferred_element_type=jnp.float32)
    r = r + jnp.dot(i, wp_ref[H1:, :], preferred_element_type=jnp.float32)
    o_ref[...] = r + bp_ref[...]


BB = 2048


def _tc_dense(uT_docs, iT_docs, wu, wi, bu, bi, wp, bp):
    grid = (B // BB,)
    return pl.pallas_call(
        _tc_dense_body,
        grid=grid,
        in_specs=[
            pl.BlockSpec((D, BB), lambda b: (0, b)),
            pl.BlockSpec((D, BB), lambda b: (0, b)),
            pl.BlockSpec((D, H1), lambda b: (0, 0)),
            pl.BlockSpec((D, H1), lambda b: (0, 0)),
            pl.BlockSpec((1, H1), lambda b: (0, 0)),
            pl.BlockSpec((1, H1), lambda b: (0, 0)),
            pl.BlockSpec((2 * H1, 1), lambda b: (0, 0)),
            pl.BlockSpec((1, 1), lambda b: (0, 0)),
        ],
        out_specs=pl.BlockSpec((BB, 1), lambda b: (b, 0)),
        out_shape=jax.ShapeDtypeStruct((B, 1), jnp.float32),
    )(uT_docs, iT_docs, wu, wi, bu, bi, wp, bp)


def kernel(batch_uid, batch_iid, uid_userDoc, iid_itemDoc, userFC_W, userFC_b,
           itemFC_W, itemFC_b, pred_W, pred_b):
    uid = batch_uid.astype(jnp.int32)
    iid = batch_iid.astype(jnp.int32)
    uT_docs, iT_docs = _make_sc_gather()(uid, iid, uid_userDoc.T,
                                         iid_itemDoc.T)
    out = _tc_dense(uT_docs, iT_docs, userFC_W, itemFC_W,
                    userFC_b.reshape(1, H1), itemFC_b.reshape(1, H1),
                    pred_W, pred_b.reshape(1, 1))
    return out


# speedup vs baseline: 1.8434x; 1.8434x over previous
import functools

import jax
import jax.numpy as jnp
from jax import lax
from jax.experimental import pallas as pl
from jax.experimental.pallas import tpu as pltpu
from jax.experimental.pallas import tpu_sc as plsc

B = 16384
D = 500
V = 100000
H1 = 64
NC = 2
NS = 16
NW = NC * NS
KMAX = (D + NW - 1) // NW
OC = 4096
NOC = B // OC


@functools.cache
def _make_sc_gather():
    mesh = plsc.VectorSubcoreMesh(core_axis_name="c", subcore_axis_name="s")

    @functools.partial(
        pl.kernel,
        mesh=mesh,
        out_type=(
            jax.ShapeDtypeStruct((D, B), jnp.float32),
            jax.ShapeDtypeStruct((D, B), jnp.float32),
        ),
        scratch_types=[
            pltpu.VMEM((B,), jnp.int32),
            pltpu.VMEM((V,), jnp.float32),
            pltpu.VMEM((2, OC), jnp.float32),
            pltpu.SemaphoreType.DMA,
            pltpu.SemaphoreType.DMA,
            pltpu.SemaphoreType.DMA,
        ],
        compiler_params=pltpu.CompilerParams(needs_layout_passes=False),
    )
    def _sc_gather(uid_hbm, iid_hbm, utabT_hbm, itabT_hbm, uoutT_hbm,
                   ioutT_hbm, idx_v, row_v, out_v, rsem, osem, xsem):
        wid = lax.axis_index("s") * NC + lax.axis_index("c")

        def one_table(idx_hbm, tabT_hbm, outT_hbm):
            pltpu.sync_copy(idx_hbm, idx_v)
            for k in range(KMAX):
                d = wid + NW * k

                @pl.when(d < D)
                def _():
                    pltpu.async_copy(tabT_hbm.at[d], row_v, rsem).wait()
                    for c in range(NOC):
                        buf = c % 2
                        if c >= 2:
                            pltpu.make_async_copy(
                                out_v.at[buf], outT_hbm.at[0, pl.ds(0, OC)],
                                osem).wait()

                        @plsc.parallel_loop(0, OC // 16, unroll=8)
                        def _gather16(v):
                            iv = idx_v[pl.ds(c * OC + v * 16, 16)]
                            out_v[buf, pl.ds(v * 16, 16)] = plsc.load_gather(
                                row_v, [iv])
                        pltpu.async_copy(
                            out_v.at[buf], outT_hbm.at[d, pl.ds(c * OC, OC)],
                            osem)
                    pltpu.make_async_copy(
                        out_v.at[0], outT_hbm.at[0, pl.ds(0, OC)], osem).wait()
                    pltpu.make_async_copy(
                        out_v.at[1], outT_hbm.at[0, pl.ds(0, OC)], osem).wait()

        one_table(uid_hbm, utabT_hbm, uoutT_hbm)
        one_table(iid_hbm, itabT_hbm, ioutT_hbm)

    return _sc_gather


def _tc_dense_body(u_ref, i_ref, wu_ref, wi_ref, bu_ref, bi_ref, wp_ref,
                   bp_ref, o_ref):
    dn = (((0,), (0,)), ((), ()))
    u = lax.dot_general(u_ref[...], wu_ref[...], dn,
                        preferred_element_type=jnp.float32)
    u = jnp.maximum(u + bu_ref[...], 0.0)
    i = lax.dot_general(i_ref[...], wi_ref[...], dn,
                        preferred_element_type=jnp.float32)
    i = jnp.maximum(i + bi_ref[...], 0.0)
    r = jnp.dot(u, wp_ref[:H1, :], preferred_element_type=jnp.float32)
    r = r + jnp.dot(i, wp_ref[H1:, :], preferred_element_type=jnp.float32)
    o_ref[...] = r + bp_ref[...]


BB = 2048


def _tc_dense(uT_docs, iT_docs, wu, wi, bu, bi, wp, bp):
    grid = (B // BB,)
    return pl.pallas_call(
        _tc_dense_body,
        grid=grid,
        in_specs=[
            pl.BlockSpec((D, BB), lambda b: (0, b)),
            pl.BlockSpec((D, BB), lambda b: (0, b)),
            pl.BlockSpec((D, H1), lambda b: (0, 0)),
            pl.BlockSpec((D, H1), lambda b: (0, 0)),
            pl.BlockSpec((1, H1), lambda b: (0, 0)),
            pl.BlockSpec((1, H1), lambda b: (0, 0)),
            pl.BlockSpec((2 * H1, 1), lambda b: (0, 0)),
            pl.BlockSpec((1, 1), lambda b: (0, 0)),
        ],
        out_specs=pl.BlockSpec((BB, 1), lambda b: (b, 0)),
        out_shape=jax.ShapeDtypeStruct((B, 1), jnp.float32),
    )(uT_docs, iT_docs, wu, wi, bu, bi, wp, bp)


def kernel(batch_uid, batch_iid, uid_userDoc, iid_itemDoc, userFC_W, userFC_b,
           itemFC_W, itemFC_b, pred_W, pred_b):
    uid = batch_uid.astype(jnp.int32)
    iid = batch_iid.astype(jnp.int32)
    uT_docs, iT_docs = _make_sc_gather()(uid, iid, uid_userDoc.T,
                                         iid_itemDoc.T)
    out = _tc_dense(uT_docs, iT_docs, userFC_W, itemFC_W,
                    userFC_b.reshape(1, H1), itemFC_b.reshape(1, H1),
                    pred_W, pred_b.reshape(1, 1))
    return out


# speedup vs baseline: 1.9153x; 1.0390x over previous
import functools

import jax
import jax.numpy as jnp
from jax import lax
from jax.experimental import pallas as pl
from jax.experimental.pallas import tpu as pltpu
from jax.experimental.pallas import tpu_sc as plsc

B = 16384
D = 500
V = 100000
H1 = 64
NC = 2
NS = 16
NW = NC * NS
KMAX = (D + NW - 1) // NW
OC = 4096
NOC = B // OC


@functools.cache
def _make_sc_gather():
    mesh = plsc.VectorSubcoreMesh(core_axis_name="c", subcore_axis_name="s")

    @functools.partial(
        pl.kernel,
        mesh=mesh,
        out_type=(
            jax.ShapeDtypeStruct((D, B), jnp.float32),
            jax.ShapeDtypeStruct((D, B), jnp.float32),
        ),
        scratch_types=[
            pltpu.VMEM((B,), jnp.int32),
            pltpu.VMEM((V,), jnp.float32),
            pltpu.VMEM((2, OC), jnp.float32),
            pltpu.SemaphoreType.DMA,
            pltpu.SemaphoreType.DMA,
            pltpu.SemaphoreType.DMA,
        ],
        compiler_params=pltpu.CompilerParams(needs_layout_passes=False),
    )
    def _sc_gather(uid_hbm, iid_hbm, utabT_hbm, itabT_hbm, uoutT_hbm,
                   ioutT_hbm, idx_v, row_v, out_v, rsem, osem, xsem):
        wid = lax.axis_index("s") * NC + lax.axis_index("c")

        def one_table(idx_hbm, tabT_hbm, outT_hbm):
            pltpu.sync_copy(idx_hbm, idx_v)

            def per_d(k, _):
                d = wid + NW * k

                @pl.when(d < D)
                def _():
                    pltpu.async_copy(tabT_hbm.at[d], row_v, rsem).wait()
                    for c in range(NOC):
                        buf = c % 2
                        if c >= 2:
                            pltpu.make_async_copy(
                                out_v.at[buf], outT_hbm.at[0, pl.ds(0, OC)],
                                osem).wait()

                        @plsc.parallel_loop(0, OC // 16, unroll=16)
                        def _gather16(v):
                            iv = idx_v[pl.ds(c * OC + v * 16, 16)]
                            out_v[buf, pl.ds(v * 16, 16)] = plsc.load_gather(
                                row_v, [iv])
                        pltpu.async_copy(
                            out_v.at[buf], outT_hbm.at[d, pl.ds(c * OC, OC)],
                            osem)
                    pltpu.make_async_copy(
                        out_v.at[0], outT_hbm.at[0, pl.ds(0, OC)], osem).wait()
                    pltpu.make_async_copy(
                        out_v.at[1], outT_hbm.at[0, pl.ds(0, OC)], osem).wait()

                return 0

            lax.fori_loop(0, KMAX, per_d, 0)

        one_table(uid_hbm, utabT_hbm, uoutT_hbm)
        one_table(iid_hbm, itabT_hbm, ioutT_hbm)

    return _sc_gather


def _tc_dense_body(u_ref, i_ref, wu_ref, wi_ref, bu_ref, bi_ref, wp_ref,
                   bp_ref, o_ref):
    dn = (((0,), (0,)), ((), ()))
    u = lax.dot_general(u_ref[...], wu_ref[...], dn,
                        preferred_element_type=jnp.float32)
    u = jnp.maximum(u + bu_ref[...], 0.0)
    i = lax.dot_general(i_ref[...], wi_ref[...], dn,
                        preferred_element_type=jnp.float32)
    i = jnp.maximum(i + bi_ref[...], 0.0)
    r = jnp.dot(u, wp_ref[:H1, :], preferred_element_type=jnp.float32)
    r = r + jnp.dot(i, wp_ref[H1:, :], preferred_element_type=jnp.float32)
    o_ref[...] = r + bp_ref[...]


BB = 2048


def _tc_dense(uT_docs, iT_docs, wu, wi, bu, bi, wp, bp):
    grid = (B // BB,)
    return pl.pallas_call(
        _tc_dense_body,
        grid=grid,
        in_specs=[
            pl.BlockSpec((D, BB), lambda b: (0, b)),
            pl.BlockSpec((D, BB), lambda b: (0, b)),
            pl.BlockSpec((D, H1), lambda b: (0, 0)),
            pl.BlockSpec((D, H1), lambda b: (0, 0)),
            pl.BlockSpec((1, H1), lambda b: (0, 0)),
            pl.BlockSpec((1, H1), lambda b: (0, 0)),
            pl.BlockSpec((2 * H1, 1), lambda b: (0, 0)),
            pl.BlockSpec((1, 1), lambda b: (0, 0)),
        ],
        out_specs=pl.BlockSpec((BB, 1), lambda b: (b, 0)),
        out_shape=jax.ShapeDtypeStruct((B, 1), jnp.float32),
    )(uT_docs, iT_docs, wu, wi, bu, bi, wp, bp)


def kernel(batch_uid, batch_iid, uid_userDoc, iid_itemDoc, userFC_W, userFC_b,
           itemFC_W, itemFC_b, pred_W, pred_b):
    uid = batch_uid.astype(jnp.int32)
    iid = batch_iid.astype(jnp.int32)
    uT_docs, iT_docs = _make_sc_gather()(uid, iid, uid_userDoc.T,
                                         iid_itemDoc.T)
    out = _tc_dense(uT_docs, iT_docs, userFC_W, itemFC_W,
                    userFC_b.reshape(1, H1), itemFC_b.reshape(1, H1),
                    pred_W, pred_b.reshape(1, 1))
    return out
